# baseline (speedup 1.0000x reference)
import functools

import jax
import jax.numpy as jnp
from jax import lax
from jax.experimental import pallas as pl
from jax.experimental.pallas import tpu as pltpu
from jax.experimental.pallas import tpu_sc as plsc

_NUM_SAMPLES = 64
_SIGMA = 0.05
_DIM = 64
_ROWS = 4096

_N_SC_ROWS = 1024
_N_TC_ROWS = _ROWS - _N_SC_ROWS
_ROW_BLOCK = 256
_N_TILES = 32
_CW = _N_SC_ROWS // _N_TILES
_SCALE = 1.0 / (_NUM_SAMPLES * _DIM)


@functools.cache
def _noise_consts():
    def make():
        n = jax.random.gumbel(
            jax.random.key(42), (_NUM_SAMPLES, _ROWS, _DIM), dtype=jnp.float32
        )
        pn = _SIGMA * n
        h = _N_TC_ROWS // 2
        pnt_a = jnp.swapaxes(pn[:, :h, :], 1, 2)
        pnt_b = jnp.swapaxes(pn[:, h:_N_TC_ROWS, :], 1, 2)
        sc = pn[:, _N_TC_ROWS:, :].reshape(_NUM_SAMPLES, _N_TILES, _CW, _DIM)
        pn_sc = sc.transpose(1, 0, 3, 2)
        return pnt_a, pnt_b, pn_sc

    return jax.jit(make)()


def _tc_rank_kernel(xt_ref, pnt_ref, o_ref):
    xt = xt_ref[...]

    def body(s, acc):
        vt = xt + pnt_ref[s]
        for b in range(_DIM):
            acc = acc + (vt > vt[b : b + 1, :]).astype(jnp.float32)
        return acc

    acc = lax.fori_loop(
        0, _NUM_SAMPLES, body, jnp.zeros((_DIM, xt.shape[1]), jnp.float32)
    )
    o_ref[...] = jnp.clip(acc * _SCALE, 0.0, 1.0)


def _tc_call(xt, pnt):
    n = xt.shape[1]
    grid = (n // _ROW_BLOCK,)
    return pl.pallas_call(
        _tc_rank_kernel,
        grid=grid,
        in_specs=[
            pl.BlockSpec((_DIM, _ROW_BLOCK), lambda i: (0, i)),
            pl.BlockSpec((_NUM_SAMPLES, _DIM, _ROW_BLOCK), lambda i: (0, 0, i)),
        ],
        out_specs=pl.BlockSpec((_DIM, _ROW_BLOCK), lambda i: (0, i)),
        out_shape=jax.ShapeDtypeStruct((_DIM, n), jnp.float32),
    )(xt, pnt)


def _sc_body(x_hbm, pn_hbm, out_hbm, x_v, n_v, v_v, acc_v):
    wid = lax.axis_index("c") * 16 + lax.axis_index("s")
    pltpu.sync_copy(x_hbm.at[wid], x_v)

    def zero_j(j, carry):
        for rc in range(_CW // 16):
            acc_v[j, pl.ds(rc * 16, 16)] = jnp.zeros((16,), jnp.float32)
        return carry

    lax.fori_loop(0, _DIM, zero_j, 0)

    def s_body(s, carry):
        pltpu.sync_copy(pn_hbm.at[wid, s], n_v)
        for rc in range(_CW // 16):
            sl = pl.ds(rc * 16, 16)
            for k in range(_DIM):
                v_v[k, :] = x_v[k, sl] + n_v[k, sl]

            def j_body(j, c):
                bj = v_v[j, :]
                parts = [jnp.zeros((16,), jnp.float32) for _ in range(8)]
                for k in range(_DIM):
                    parts[k % 8] = parts[k % 8] + jnp.where(
                        v_v[k, :] < bj, 1.0, 0.0
                    )
                while len(parts) > 1:
                    parts = [a + b for a, b in zip(parts[::2], parts[1::2])]
                cnt = parts[0]
                acc_v[j, sl] = acc_v[j, sl] + cnt
                return c

            lax.fori_loop(0, _DIM, j_body, 0)
        return carry

    lax.fori_loop(0, _NUM_SAMPLES, s_body, 0)

    def fin_j(j, carry):
        for rc in range(_CW // 16):
            sl = pl.ds(rc * 16, 16)
            acc_v[j, sl] = jnp.clip(acc_v[j, sl] * _SCALE, 0.0, 1.0)
        return carry

    lax.fori_loop(0, _DIM, fin_j, 0)
    pltpu.sync_copy(acc_v, out_hbm.at[wid])


def _sc_call(x_sc):
    mesh = plsc.VectorSubcoreMesh(
        core_axis_name="c", subcore_axis_name="s", num_cores=2, num_subcores=16
    )
    n_pairs = _N_SC_ROWS * _NUM_SAMPLES * _DIM * _DIM
    return pl.kernel(
        _sc_body,
        out_type=jax.ShapeDtypeStruct((_N_TILES, _DIM, _CW), jnp.float32),
        mesh=mesh,
        cost_estimate=pl.CostEstimate(
            flops=3 * n_pairs,
            transcendentals=0,
            bytes_accessed=4 * _N_SC_ROWS * _DIM * (_NUM_SAMPLES + 3),
        ),
        scratch_types=[
            pltpu.VMEM((_DIM, _CW), jnp.float32),
            pltpu.VMEM((_DIM, _CW), jnp.float32),
            pltpu.VMEM((_DIM, 16), jnp.float32),
            pltpu.VMEM((_DIM, _CW), jnp.float32),
        ],
    )(x_sc, _noise_consts()[2])


def kernel(X):
    x_sc = X[_N_TC_ROWS:].reshape(_N_TILES, _CW, _DIM).transpose(0, 2, 1)
    out_sc = _sc_call(x_sc)
    pnt_a, pnt_b, _ = _noise_consts()
    h = _N_TC_ROWS // 2
    xt = X[:_N_TC_ROWS].T
    out_a = _tc_call(xt[:, :h], pnt_a)
    out_b = _tc_call(xt[:, h:], pnt_b)
    return jnp.concatenate(
        [
            out_a.T,
            out_b.T,
            out_sc.transpose(0, 2, 1).reshape(_N_SC_ROWS, _DIM),
        ],
        axis=0,
    )


# speedup vs baseline: 1.1283x; 1.1283x over previous
import functools

import jax
import jax.numpy as jnp
from jax import lax
from jax.experimental import pallas as pl
from jax.experimental.pallas import tpu as pltpu
from jax.experimental.pallas import tpu_sc as plsc

_NUM_SAMPLES = 64
_SIGMA = 0.05
_DIM = 64
_ROWS = 4096

_N_SC_ROWS = 512
_N_TC_ROWS = _ROWS - _N_SC_ROWS
_ROW_BLOCK = 128
_N_TILES = 32
_CW = _N_SC_ROWS // _N_TILES
_SCALE = 1.0 / (_NUM_SAMPLES * _DIM)


@functools.cache
def _noise_consts():
    def make():
        n = jax.random.gumbel(
            jax.random.key(42), (_NUM_SAMPLES, _ROWS, _DIM), dtype=jnp.float32
        )
        pn = _SIGMA * n
        h = _N_TC_ROWS // 2
        pnt_a = jnp.swapaxes(pn[:, :h, :], 1, 2)
        pnt_b = jnp.swapaxes(pn[:, h:_N_TC_ROWS, :], 1, 2)
        sc = pn[:, _N_TC_ROWS:, :].reshape(_NUM_SAMPLES, _N_TILES, _CW, _DIM)
        pn_sc = sc.transpose(1, 0, 3, 2)
        return pnt_a, pnt_b, pn_sc

    return jax.jit(make)()


def _tc_rank_kernel(xt_ref, pnt_ref, o_ref):
    xt = xt_ref[...]

    def body(s, acc):
        vt = xt + pnt_ref[s]
        for b in range(_DIM):
            acc = acc + (vt > vt[b : b + 1, :]).astype(jnp.float32)
        return acc

    acc = lax.fori_loop(
        0, _NUM_SAMPLES, body, jnp.zeros((_DIM, xt.shape[1]), jnp.float32)
    )
    o_ref[...] = jnp.clip(acc * _SCALE, 0.0, 1.0)


def _tc_call(xt, pnt):
    n = xt.shape[1]
    grid = (n // _ROW_BLOCK,)
    return pl.pallas_call(
        _tc_rank_kernel,
        grid=grid,
        in_specs=[
            pl.BlockSpec((_DIM, _ROW_BLOCK), lambda i: (0, i)),
            pl.BlockSpec((_NUM_SAMPLES, _DIM, _ROW_BLOCK), lambda i: (0, 0, i)),
        ],
        out_specs=pl.BlockSpec((_DIM, _ROW_BLOCK), lambda i: (0, i)),
        out_shape=jax.ShapeDtypeStruct((_DIM, n), jnp.float32),
    )(xt, pnt)


def _sc_body(x_hbm, pn_hbm, out_hbm, x_v, n_v, v_v, acc_v):
    wid = lax.axis_index("c") * 16 + lax.axis_index("s")
    pltpu.sync_copy(x_hbm.at[wid], x_v)

    def zero_j(j, carry):
        for rc in range(_CW // 16):
            acc_v[j, pl.ds(rc * 16, 16)] = jnp.zeros((16,), jnp.float32)
        return carry

    lax.fori_loop(0, _DIM, zero_j, 0)

    def s_body(s, carry):
        pltpu.sync_copy(pn_hbm.at[wid, s], n_v)
        for rc in range(_CW // 16):
            sl = pl.ds(rc * 16, 16)
            for k in range(_DIM):
                v_v[k, :] = x_v[k, sl] + n_v[k, sl]

            def j_body(j, c):
                bj = v_v[j, :]
                parts = [jnp.zeros((16,), jnp.float32) for _ in range(8)]
                for k in range(_DIM):
                    parts[k % 8] = parts[k % 8] + jnp.where(
                        v_v[k, :] < bj, 1.0, 0.0
                    )
                while len(parts) > 1:
                    parts = [a + b for a, b in zip(parts[::2], parts[1::2])]
                cnt = parts[0]
                acc_v[j, sl] = acc_v[j, sl] + cnt
                return c

            lax.fori_loop(0, _DIM, j_body, 0)
        return carry

    lax.fori_loop(0, _NUM_SAMPLES, s_body, 0)

    def fin_j(j, carry):
        for rc in range(_CW // 16):
            sl = pl.ds(rc * 16, 16)
            acc_v[j, sl] = jnp.clip(acc_v[j, sl] * _SCALE, 0.0, 1.0)
        return carry

    lax.fori_loop(0, _DIM, fin_j, 0)
    pltpu.sync_copy(acc_v, out_hbm.at[wid])


def _sc_call(x_sc):
    mesh = plsc.VectorSubcoreMesh(
        core_axis_name="c", subcore_axis_name="s", num_cores=2, num_subcores=16
    )
    n_pairs = _N_SC_ROWS * _NUM_SAMPLES * _DIM * _DIM
    return pl.kernel(
        _sc_body,
        out_type=jax.ShapeDtypeStruct((_N_TILES, _DIM, _CW), jnp.float32),
        mesh=mesh,
        cost_estimate=pl.CostEstimate(
            flops=3 * n_pairs,
            transcendentals=0,
            bytes_accessed=4 * _N_SC_ROWS * _DIM * (_NUM_SAMPLES + 3),
        ),
        scratch_types=[
            pltpu.VMEM((_DIM, _CW), jnp.float32),
            pltpu.VMEM((_DIM, _CW), jnp.float32),
            pltpu.VMEM((_DIM, 16), jnp.float32),
            pltpu.VMEM((_DIM, _CW), jnp.float32),
        ],
    )(x_sc, _noise_consts()[2])


def kernel(X):
    x_sc = X[_N_TC_ROWS:].reshape(_N_TILES, _CW, _DIM).transpose(0, 2, 1)
    out_sc = _sc_call(x_sc)
    pnt_a, pnt_b, _ = _noise_consts()
    h = _N_TC_ROWS // 2
    xt = X[:_N_TC_ROWS].T
    out_a = _tc_call(xt[:, :h], pnt_a)
    out_b = _tc_call(xt[:, h:], pnt_b)
    return jnp.concatenate(
        [
            out_a.T,
            out_b.T,
            out_sc.transpose(0, 2, 1).reshape(_N_SC_ROWS, _DIM),
        ],
        axis=0,
    )


# speedup vs baseline: 1.2025x; 1.0657x over previous
import functools

import jax
import jax.numpy as jnp
from jax import lax
from jax.experimental import pallas as pl
from jax.experimental.pallas import tpu as pltpu
from jax.experimental.pallas import tpu_sc as plsc

_NUM_SAMPLES = 64
_SIGMA = 0.05
_DIM = 64
_ROWS = 4096

_N_SC_ROWS = 512
_N_TC_ROWS = _ROWS - _N_SC_ROWS
_ROW_BLOCK = 512
_N_TILES = 32
_CW = _N_SC_ROWS // _N_TILES
_SCALE = 1.0 / (_NUM_SAMPLES * _DIM)


@functools.cache
def _noise_consts():
    def make():
        n = jax.random.gumbel(
            jax.random.key(42), (_NUM_SAMPLES, _ROWS, _DIM), dtype=jnp.float32
        )
        pn = _SIGMA * n
        h = _N_TC_ROWS // 2
        pnt_a = jnp.swapaxes(pn[:, :h, :], 1, 2)
        pnt_b = jnp.swapaxes(pn[:, h:_N_TC_ROWS, :], 1, 2)
        sc = pn[:, _N_TC_ROWS:, :].reshape(_NUM_SAMPLES, _N_TILES, _CW, _DIM)
        pn_sc = sc.transpose(1, 0, 3, 2)
        return pnt_a, pnt_b, pn_sc

    return jax.jit(make)()


def _tc_rank_kernel(xt_ref, pnt_ref, o_ref):
    xt = xt_ref[...]

    def body(s, acc):
        vt = xt + pnt_ref[s]
        for b in range(_DIM):
            acc = acc + (vt > vt[b : b + 1, :]).astype(jnp.float32)
        return acc

    acc = lax.fori_loop(
        0, _NUM_SAMPLES, body, jnp.zeros((_DIM, xt.shape[1]), jnp.float32)
    )
    o_ref[...] = jnp.clip(acc * _SCALE, 0.0, 1.0)


def _tc_call(xt, pnt):
    n = xt.shape[1]
    grid = (n // _ROW_BLOCK,)
    return pl.pallas_call(
        _tc_rank_kernel,
        grid=grid,
        in_specs=[
            pl.BlockSpec((_DIM, _ROW_BLOCK), lambda i: (0, i)),
            pl.BlockSpec((_NUM_SAMPLES, _DIM, _ROW_BLOCK), lambda i: (0, 0, i)),
        ],
        out_specs=pl.BlockSpec((_DIM, _ROW_BLOCK), lambda i: (0, i)),
        out_shape=jax.ShapeDtypeStruct((_DIM, n), jnp.float32),
    )(xt, pnt)


def _sc_body(x_hbm, pn_hbm, out_hbm, x_v, n_v, v_v, acc_v):
    wid = lax.axis_index("c") * 16 + lax.axis_index("s")
    pltpu.sync_copy(x_hbm.at[wid], x_v)

    def zero_j(j, carry):
        for rc in range(_CW // 16):
            acc_v[j, pl.ds(rc * 16, 16)] = jnp.zeros((16,), jnp.float32)
        return carry

    lax.fori_loop(0, _DIM, zero_j, 0)

    def s_body(s, carry):
        pltpu.sync_copy(pn_hbm.at[wid, s], n_v)
        for rc in range(_CW // 16):
            sl = pl.ds(rc * 16, 16)
            for k in range(_DIM):
                v_v[k, :] = x_v[k, sl] + n_v[k, sl]

            def j_body(j, c):
                bj = v_v[j, :]
                parts = [jnp.zeros((16,), jnp.float32) for _ in range(8)]
                for k in range(_DIM):
                    parts[k % 8] = parts[k % 8] + jnp.where(
                        v_v[k, :] < bj, 1.0, 0.0
                    )
                while len(parts) > 1:
                    parts = [a + b for a, b in zip(parts[::2], parts[1::2])]
                cnt = parts[0]
                acc_v[j, sl] = acc_v[j, sl] + cnt
                return c

            lax.fori_loop(0, _DIM, j_body, 0)
        return carry

    lax.fori_loop(0, _NUM_SAMPLES, s_body, 0)

    def fin_j(j, carry):
        for rc in range(_CW // 16):
            sl = pl.ds(rc * 16, 16)
            acc_v[j, sl] = jnp.clip(acc_v[j, sl] * _SCALE, 0.0, 1.0)
        return carry

    lax.fori_loop(0, _DIM, fin_j, 0)
    pltpu.sync_copy(acc_v, out_hbm.at[wid])


def _sc_call(x_sc):
    mesh = plsc.VectorSubcoreMesh(
        core_axis_name="c", subcore_axis_name="s", num_cores=2, num_subcores=16
    )
    n_pairs = _N_SC_ROWS * _NUM_SAMPLES * _DIM * _DIM
    return pl.kernel(
        _sc_body,
        out_type=jax.ShapeDtypeStruct((_N_TILES, _DIM, _CW), jnp.float32),
        mesh=mesh,
        cost_estimate=pl.CostEstimate(
            flops=3 * n_pairs,
            transcendentals=0,
            bytes_accessed=4 * _N_SC_ROWS * _DIM * (_NUM_SAMPLES + 3),
        ),
        scratch_types=[
            pltpu.VMEM((_DIM, _CW), jnp.float32),
            pltpu.VMEM((_DIM, _CW), jnp.float32),
            pltpu.VMEM((_DIM, 16), jnp.float32),
            pltpu.VMEM((_DIM, _CW), jnp.float32),
        ],
    )(x_sc, _noise_consts()[2])


def kernel(X):
    x_sc = X[_N_TC_ROWS:].reshape(_N_TILES, _CW, _DIM).transpose(0, 2, 1)
    out_sc = _sc_call(x_sc)
    pnt_a, pnt_b, _ = _noise_consts()
    h = _N_TC_ROWS // 2
    xt = X[:_N_TC_ROWS].T
    out_a = _tc_call(xt[:, :h], pnt_a)
    out_b = _tc_call(xt[:, h:], pnt_b)
    return jnp.concatenate(
        [
            out_a.T,
            out_b.T,
            out_sc.transpose(0, 2, 1).reshape(_N_SC_ROWS, _DIM),
        ],
        axis=0,
    )
